# Initial kernel scaffold; baseline (speedup 1.0000x reference)
import functools

import jax
import jax.numpy as jnp
from jax import lax
from jax.experimental import pallas as pl
from jax.experimental.pallas import tpu as pltpu
from jax.experimental.pallas import tpu_sc as plsc

_N = 10000
_D = 128
_H = 128
_C = 40
_E = 320000

_NC = 2
_NS = 16
_NW = _NC * _NS
_LANES = 16

_NPAD = 10240
_RPT = _NPAD // _NS
_CH = 80
_EPAD = _NW * _CH * 128
_CPAD = 48

_MESH = plsc.VectorSubcoreMesh(
    core_axis_name="c", subcore_axis_name="s", num_cores=_NC, num_subcores=_NS
)


@functools.partial(
    pl.kernel,
    out_type=jax.ShapeDtypeStruct((_NC, _NPAD), jnp.float32),
    mesh=_MESH,
    scratch_types=[
        pltpu.VMEM((_CH, 128), jnp.int32),
        pltpu.VMEM((_NPAD,), jnp.float32),
        pltpu.VMEM((_RPT,), jnp.float32),
        pltpu.VMEM((_RPT,), jnp.float32),
        pltpu.VMEM_SHARED((_NS, _NPAD), jnp.float32),
    ],
)
def _deg_kernel(dst_hbm, deg_hbm, dstb, hist, red, tmp, hist_sh):
    c = lax.axis_index("c")
    s = lax.axis_index("s")
    wid = s * _NC + c
    pltpu.sync_copy(dst_hbm.at[wid], dstb)

    zero16 = jnp.zeros((_LANES,), jnp.float32)

    def zbody(i, u):
        hist[pl.ds(i * _LANES, _LANES)] = zero16
        return u

    lax.fori_loop(0, _NPAD // _LANES, zbody, 0)

    one16 = jnp.ones((_LANES,), jnp.float32)

    def sbody(j, u):
        for v in range(128 // _LANES):
            idx = dstb[j, pl.ds(v * _LANES, _LANES)]
            plsc.addupdate_scatter(hist, [idx], one16)
        return u

    lax.fori_loop(0, _CH, sbody, 0)

    pltpu.sync_copy(hist, hist_sh.at[s])
    plsc.subcore_barrier()

    row = pl.ds(s * _RPT, _RPT)
    pltpu.sync_copy(hist_sh.at[0, row], red)

    def rbody(k, u):
        pltpu.sync_copy(hist_sh.at[k, row], tmp)

        def abody(i, uu):
            sl = pl.ds(i * _LANES, _LANES)
            red[sl] = red[sl] + tmp[sl]
            return uu

        lax.fori_loop(0, _RPT // _LANES, abody, 0)
        return u

    lax.fori_loop(1, _NS, rbody, 0)
    pltpu.sync_copy(red, deg_hbm.at[c, row])


def _make_spmm(width):
    @functools.partial(
        pl.kernel,
        out_type=jax.ShapeDtypeStruct((_NC, _NPAD, width), jnp.float32),
        mesh=_MESH,
        scratch_types=[
            pltpu.VMEM((_CH, 128), jnp.int32),
            pltpu.VMEM((_CH, 128), jnp.int32),
            pltpu.VMEM((2, 128, width), jnp.float32),
            pltpu.VMEM_SHARED((_NPAD, width), jnp.float32),
            pltpu.SemaphoreType.DMA,
            pltpu.SemaphoreType.DMA,
        ],
    )
    def _spmm(y_hbm, src_hbm, dst_hbm, z_hbm, srcb, dstb, rows, acc, sem0, sem1):
        c = lax.axis_index("c")
        s = lax.axis_index("s")
        wid = s * _NC + c
        pltpu.sync_copy(src_hbm.at[wid], srcb)
        pltpu.sync_copy(dst_hbm.at[wid], dstb)

        row = pl.ds(s * _RPT, _RPT)
        pltpu.sync_copy(y_hbm.at[row], acc.at[row])
        plsc.subcore_barrier()

        sems = (sem0, sem1)
        for b in range(2):
            pltpu.async_copy(y_hbm.at[srcb.at[b]], rows.at[b], sems[b])

        def body(i, u):
            j0 = i * 2
            for b in range(2):
                j = j0 + b
                pltpu.make_async_copy(y_hbm.at[srcb.at[j]], rows.at[b], sems[b]).wait()
                pltpu.sync_copy(rows.at[b], acc.at[dstb.at[j]], add=True)
                pltpu.async_copy(y_hbm.at[srcb.at[j + 2]], rows.at[b], sems[b])
            return u

        lax.fori_loop(0, (_CH - 2) // 2, body, 0)

        for b in range(2):
            j = _CH - 2 + b
            pltpu.make_async_copy(y_hbm.at[srcb.at[j]], rows.at[b], sems[b]).wait()
            pltpu.sync_copy(rows.at[b], acc.at[dstb.at[j]], add=True)

        plsc.subcore_barrier()
        pltpu.sync_copy(acc.at[row], z_hbm.at[c, row])

    return _spmm


_spmm_h = _make_spmm(_H)
_spmm_c = _make_spmm(_CPAD)


def _tc_pre_body(deg_ref, x_ref, w_ref, y_ref, dinv_ref):
    deg = deg_ref[0, :] + deg_ref[1, :] + 1.0
    dinv = lax.rsqrt(deg)
    dinv_ref[...] = dinv
    y_ref[...] = jnp.dot(
        x_ref[...] * dinv[:, None], w_ref[...], preferred_element_type=jnp.float32
    )


_tc_pre = pl.pallas_call(
    _tc_pre_body,
    out_shape=(
        jax.ShapeDtypeStruct((_NPAD, _H), jnp.float32),
        jax.ShapeDtypeStruct((_NPAD,), jnp.float32),
    ),
)


def _tc_mid_common(z_ref, yprev_ref, res_ref, dinv_ref, b_ref, w_ref,
                   h_ref, ynext_ref):
    dinv = dinv_ref[...]
    z = z_ref[0] + z_ref[1] - yprev_ref[...]
    t = z * dinv[:, None] + b_ref[...][None, :]
    if res_ref is not None:
        t = t + res_ref[...]
    h = jnp.where(t > 0, t, jnp.expm1(t))
    h_ref[...] = h
    ynext_ref[...] = jnp.dot(
        h * dinv[:, None], w_ref[...], preferred_element_type=jnp.float32
    )


def _tc_mid1_body(z_ref, yprev_ref, dinv_ref, b_ref, w_ref, h_ref, ynext_ref):
    _tc_mid_common(z_ref, yprev_ref, None, dinv_ref, b_ref, w_ref, h_ref, ynext_ref)


def _tc_mid2_body(z_ref, yprev_ref, res_ref, dinv_ref, b_ref, w_ref, h_ref,
                  ynext_ref):
    _tc_mid_common(z_ref, yprev_ref, res_ref, dinv_ref, b_ref, w_ref, h_ref,
                   ynext_ref)


_tc_mid1 = pl.pallas_call(
    _tc_mid1_body,
    out_shape=(
        jax.ShapeDtypeStruct((_NPAD, _H), jnp.float32),
        jax.ShapeDtypeStruct((_NPAD, _H), jnp.float32),
    ),
)

_tc_mid2 = pl.pallas_call(
    _tc_mid2_body,
    out_shape=(
        jax.ShapeDtypeStruct((_NPAD, _H), jnp.float32),
        jax.ShapeDtypeStruct((_NPAD, _CPAD), jnp.float32),
    ),
)


def _tc_post_body(z_ref, yprev_ref, dinv_ref, b_ref, out_ref):
    z = z_ref[0] + z_ref[1] - yprev_ref[...]
    out_ref[...] = z * dinv_ref[...][:, None] + b_ref[...][None, :]


_tc_post = pl.pallas_call(
    _tc_post_body,
    out_shape=jax.ShapeDtypeStruct((_NPAD, _CPAD), jnp.float32),
)


def kernel(x, edge_index, W1, b1, W2, b2, W3, b3):
    src = edge_index[0].astype(jnp.int32)
    dst = edge_index[1].astype(jnp.int32)
    fill = jnp.full((_EPAD - _E,), _N, jnp.int32)
    srcp = jnp.concatenate([src, fill]).reshape(_NW, _CH, 128)
    dstp = jnp.concatenate([dst, fill]).reshape(_NW, _CH, 128)
    xp = jnp.zeros((_NPAD, _D), jnp.float32).at[:_N].set(x)
    w3p = jnp.zeros((_H, _CPAD), jnp.float32).at[:, :_C].set(W3)
    b3p = jnp.zeros((_CPAD,), jnp.float32).at[:_C].set(b3)

    deg = _deg_kernel(dstp)
    y1, dinv = _tc_pre(deg, xp, W1)
    z1 = _spmm_h(y1, srcp, dstp)
    h1, y2 = _tc_mid1(z1, y1, dinv, b1, W2)
    z2 = _spmm_h(y2, srcp, dstp)
    h2, y3 = _tc_mid2(z2, y2, h1, dinv, b2, w3p)
    z3 = _spmm_c(y3, srcp, dstp)
    out = _tc_post(z3, y3, dinv, b3p)
    return out[:_N, :_C]


# speedup vs baseline: 7.6036x; 7.6036x over previous
import functools

import jax
import jax.numpy as jnp
from jax import lax
from jax.experimental import pallas as pl
from jax.experimental.pallas import tpu as pltpu
from jax.experimental.pallas import tpu_sc as plsc

_N = 10000
_D = 128
_H = 128
_C = 40
_E = 320000

_NC = 2
_NS = 16
_NW = _NC * _NS
_LANES = 16

_NPAD = 10240
_RPT = _NPAD // _NS
_CW = 128
_CH = 80
_EPAD = _NW * _CH * _CW
_CPAD = 48

_MESH = plsc.VectorSubcoreMesh(
    core_axis_name="c", subcore_axis_name="s", num_cores=_NC, num_subcores=_NS
)


@functools.partial(
    pl.kernel,
    out_type=jax.ShapeDtypeStruct((_NC, _NPAD), jnp.float32),
    mesh=_MESH,
    compiler_params=pltpu.CompilerParams(needs_layout_passes=False),
    scratch_types=[
        pltpu.VMEM((_CH, _CW), jnp.int32),
        pltpu.VMEM((_NPAD,), jnp.float32),
        pltpu.VMEM((_RPT,), jnp.float32),
        pltpu.VMEM((_RPT,), jnp.float32),
        pltpu.VMEM_SHARED((_NS, _NPAD), jnp.float32),
    ],
)
def _deg_kernel(pack_hbm, deg_hbm, packb, hist, red, tmp, hist_sh):
    c = lax.axis_index("c")
    s = lax.axis_index("s")
    wid = s * _NC + c
    pltpu.sync_copy(pack_hbm.at[wid], packb)

    zero16 = jnp.zeros((_LANES,), jnp.float32)

    def zbody(i, u):
        hist[pl.ds(i * _LANES, _LANES)] = zero16
        return u

    lax.fori_loop(0, _NPAD // _LANES, zbody, 0)

    one16 = jnp.ones((_LANES,), jnp.float32)

    def sbody(j, u):
        for v in range(_CW // _LANES):
            idx = lax.shift_right_logical(packb[j, pl.ds(v * _LANES, _LANES)], 16)
            plsc.addupdate_scatter(hist, [idx], one16)
        return u

    lax.fori_loop(0, _CH, sbody, 0)

    pltpu.sync_copy(hist, hist_sh.at[s])
    plsc.subcore_barrier()

    row = pl.ds(s * _RPT, _RPT)
    pltpu.sync_copy(hist_sh.at[0, row], red)

    def rbody(k, u):
        pltpu.sync_copy(hist_sh.at[k, row], tmp)

        def abody(i, uu):
            sl = pl.ds(i * _LANES, _LANES)
            red[sl] = red[sl] + tmp[sl]
            return uu

        lax.fori_loop(0, _RPT // _LANES, abody, 0)
        return u

    lax.fori_loop(1, _NS, rbody, 0)
    pltpu.sync_copy(red, deg_hbm.at[c, row])


def _make_spmm(width):
    @functools.partial(
        pl.kernel,
        out_type=jax.ShapeDtypeStruct((_NC, _NPAD, width), jnp.float32),
        mesh=_MESH,
        compiler_params=pltpu.CompilerParams(needs_layout_passes=False),
        scratch_types=[
            pltpu.VMEM((_CH, _CW), jnp.int32),
            pltpu.VMEM((2, _CW), jnp.int32),
            pltpu.VMEM((2, _CW), jnp.int32),
            pltpu.VMEM((2, _CW, width), jnp.float32),
            pltpu.VMEM_SHARED((_NPAD, width), jnp.float32),
            pltpu.SemaphoreType.DMA,
            pltpu.SemaphoreType.DMA,
        ],
    )
    def _spmm(y_hbm, pack_hbm, z_hbm, packb, sidx, didx, rows, acc, sem0, sem1):
        c = lax.axis_index("c")
        s = lax.axis_index("s")
        wid = s * _NC + c
        pltpu.sync_copy(pack_hbm.at[wid], packb)

        row = pl.ds(s * _RPT, _RPT)
        pltpu.sync_copy(y_hbm.at[row], acc.at[row])
        plsc.subcore_barrier()

        sems = (sem0, sem1)
        mask = jnp.full((_LANES,), 0xFFFF, jnp.int32)

        def unpack(j, b):
            for v in range(_CW // _LANES):
                sl = pl.ds(v * _LANES, _LANES)
                pk = packb[j, sl]
                sidx[b, sl] = lax.bitwise_and(pk, mask)
                didx[b, sl] = lax.shift_right_logical(pk, 16)

        def gather_start(b):
            pltpu.async_copy(y_hbm.at[sidx.at[b]], rows.at[b], sems[b])

        def gather_wait(b):
            pltpu.make_async_copy(y_hbm.at[sidx.at[b]], rows.at[b], sems[b]).wait()

        def scatter(b):
            pltpu.sync_copy(rows.at[b], acc.at[didx.at[b]], add=True)

        for b in range(2):
            unpack(b, b)
            gather_start(b)

        def body(i, u):
            for b in range(2):
                j = i * 2 + b
                gather_wait(b)
                scatter(b)
                unpack(j + 2, b)
                gather_start(b)
            return u

        lax.fori_loop(0, (_CH - 2) // 2, body, 0)

        for b in range(2):
            gather_wait(b)
            scatter(b)

        plsc.subcore_barrier()
        pltpu.sync_copy(acc.at[row], z_hbm.at[c, row])

    return _spmm


_spmm_h = _make_spmm(_H)


def _tc_pre_body(deg_ref, x_ref, w_ref, y_ref, dinv_ref):
    deg = deg_ref[0, :] + deg_ref[1, :] + 1.0
    dinv = lax.rsqrt(deg)
    dinv_ref[...] = dinv
    y_ref[...] = jnp.dot(
        x_ref[...] * dinv[:, None], w_ref[...], preferred_element_type=jnp.float32
    )


_tc_pre = pl.pallas_call(
    _tc_pre_body,
    out_shape=(
        jax.ShapeDtypeStruct((_NPAD, _H), jnp.float32),
        jax.ShapeDtypeStruct((_NPAD,), jnp.float32),
    ),
)


def _tc_mid1_body(z_ref, yprev_ref, dinv_ref, b_ref, w_ref, h_ref, ynext_ref):
    dinv = dinv_ref[...]
    z = z_ref[0] + z_ref[1] - yprev_ref[...]
    t = z * dinv[:, None] + b_ref[...][None, :]
    h = jnp.where(t > 0, t, jnp.exp(jnp.minimum(t, 0.0)) - 1.0)
    h_ref[...] = h
    ynext_ref[...] = jnp.dot(
        h * dinv[:, None], w_ref[...], preferred_element_type=jnp.float32
    )


def _tc_mid2_body(z_ref, yprev_ref, res_ref, dinv_ref, b_ref, h_ref, u_ref):
    dinv = dinv_ref[...]
    z = z_ref[0] + z_ref[1] - yprev_ref[...]
    t = z * dinv[:, None] + b_ref[...][None, :] + res_ref[...]
    h = jnp.where(t > 0, t, jnp.exp(jnp.minimum(t, 0.0)) - 1.0)
    h_ref[...] = h
    u_ref[...] = h * dinv[:, None]


_tc_mid1 = pl.pallas_call(
    _tc_mid1_body,
    out_shape=(
        jax.ShapeDtypeStruct((_NPAD, _H), jnp.float32),
        jax.ShapeDtypeStruct((_NPAD, _H), jnp.float32),
    ),
)

_tc_mid2 = pl.pallas_call(
    _tc_mid2_body,
    out_shape=(
        jax.ShapeDtypeStruct((_NPAD, _H), jnp.float32),
        jax.ShapeDtypeStruct((_NPAD, _H), jnp.float32),
    ),
)


def _tc_post_body(z_ref, uprev_ref, dinv_ref, b_ref, w_ref, out_ref):
    z = z_ref[0] + z_ref[1] - uprev_ref[...]
    out_ref[...] = (
        jnp.dot(z * dinv_ref[...][:, None], w_ref[...],
                preferred_element_type=jnp.float32)
        + b_ref[...][None, :]
    )


_tc_post = pl.pallas_call(
    _tc_post_body,
    out_shape=jax.ShapeDtypeStruct((_NPAD, _CPAD), jnp.float32),
)


def kernel(x, edge_index, W1, b1, W2, b2, W3, b3):
    src = edge_index[0].astype(jnp.int32)
    dst = edge_index[1].astype(jnp.int32)
    packed = jnp.bitwise_or(src, jnp.left_shift(dst, 16))
    fillv = jnp.int32(_N | (_N << 16))
    fill = jnp.full((_EPAD - _E,), fillv, jnp.int32)
    packp = jnp.concatenate([packed, fill]).reshape(_NW, _CH, _CW)
    xp = jnp.zeros((_NPAD, _D), jnp.float32).at[:_N].set(x)
    w3p = jnp.zeros((_H, _CPAD), jnp.float32).at[:, :_C].set(W3)
    b3p = jnp.zeros((_CPAD,), jnp.float32).at[:_C].set(b3)

    deg = _deg_kernel(packp)
    y1, dinv = _tc_pre(deg, xp, W1)
    z1 = _spmm_h(y1, packp)
    h1, y2 = _tc_mid1(z1, y1, dinv, b1, W2)
    z2 = _spmm_h(y2, packp)
    h2, u3 = _tc_mid2(z2, y2, h1, dinv, b2)
    z3 = _spmm_h(u3, packp)
    out = _tc_post(z3, u3, dinv, b3p, w3p)
    return out[:_N, :_C]


# speedup vs baseline: 8.6559x; 1.1384x over previous
import functools

import jax
import jax.numpy as jnp
from jax import lax
from jax.experimental import pallas as pl
from jax.experimental.pallas import tpu as pltpu
from jax.experimental.pallas import tpu_sc as plsc

_N = 10000
_D = 128
_H = 128
_C = 40
_E = 320000

_NC = 2
_NS = 16
_NW = _NC * _NS
_LANES = 16

_NPAD = 10240
_RPT = _NPAD // _NS
_NROW = 2816
_EPAD = _NROW * 128
_DGR = _NROW // _NW
_FASTC = 0
_CHF = 136
_CHS = 24
_ROW0S = _NS * _CHF
_CPAD = 48

_MESH = plsc.VectorSubcoreMesh(
    core_axis_name="c", subcore_axis_name="s", num_cores=_NC, num_subcores=_NS
)


@functools.partial(
    pl.kernel,
    out_type=jax.ShapeDtypeStruct((_NC, _NPAD), jnp.float32),
    mesh=_MESH,
    compiler_params=pltpu.CompilerParams(needs_layout_passes=False),
    scratch_types=[
        pltpu.VMEM((_DGR, 128), jnp.int32),
        pltpu.VMEM((_NPAD,), jnp.float32),
        pltpu.VMEM((_RPT,), jnp.float32),
        pltpu.VMEM((_RPT,), jnp.float32),
        pltpu.VMEM_SHARED((_NS, _NPAD), jnp.float32),
    ],
)
def _deg_kernel(pack_hbm, deg_hbm, packb, hist, red, tmp, hist_sh):
    c = lax.axis_index("c")
    s = lax.axis_index("s")
    wid = s * _NC + c
    pltpu.sync_copy(pack_hbm.at[pl.ds(wid * _DGR, _DGR)], packb)

    zero16 = jnp.zeros((_LANES,), jnp.float32)

    def zbody(i, u):
        hist[pl.ds(i * _LANES, _LANES)] = zero16
        return u

    lax.fori_loop(0, _NPAD // _LANES, zbody, 0)

    one16 = jnp.ones((_LANES,), jnp.float32)

    def sbody(j, u):
        for v in range(128 // _LANES):
            idx = lax.shift_right_logical(packb[j, pl.ds(v * _LANES, _LANES)], 16)
            plsc.addupdate_scatter(hist, [idx], one16)
        return u

    lax.fori_loop(0, _DGR, sbody, 0)

    pltpu.sync_copy(hist, hist_sh.at[s])
    plsc.subcore_barrier()

    row = pl.ds(s * _RPT, _RPT)
    pltpu.sync_copy(hist_sh.at[0, row], red)

    def rbody(k, u):
        pltpu.sync_copy(hist_sh.at[k, row], tmp)

        def abody(i, uu):
            sl = pl.ds(i * _LANES, _LANES)
            red[sl] = red[sl] + tmp[sl]
            return uu

        lax.fori_loop(0, _RPT // _LANES, abody, 0)
        return u

    lax.fori_loop(1, _NS, rbody, 0)
    pltpu.sync_copy(red, deg_hbm.at[c, row])


def _make_spmm(width):
    @functools.partial(
        pl.kernel,
        out_type=jax.ShapeDtypeStruct((_NC, _NPAD, width), jnp.float32),
        mesh=_MESH,
        compiler_params=pltpu.CompilerParams(needs_layout_passes=False),
        scratch_types=[
            pltpu.VMEM((_CHF, 128), jnp.int32),
            pltpu.VMEM((2, 64), jnp.int32),
            pltpu.VMEM((2, 64), jnp.int32),
            pltpu.VMEM((2, 64, width), jnp.float32),
            pltpu.VMEM_SHARED((_NPAD, width), jnp.float32),
            pltpu.SemaphoreType.DMA,
            pltpu.SemaphoreType.DMA,
        ],
    )
    def _spmm(y_hbm, pack_hbm, z_hbm, packb, sidx, didx, rows, acc, sem0, sem1):
        c = lax.axis_index("c")
        s = lax.axis_index("s")
        nch = jnp.where(c == _FASTC, _CHF, _CHS)
        coff = jnp.where(c == _FASTC, s * _CHF, _ROW0S + s * _CHS)
        pltpu.sync_copy(pack_hbm.at[pl.ds(coff, _CHF)], packb)

        row = pl.ds(s * _RPT, _RPT)

        @pl.when(c == _FASTC)
        def _():
            pltpu.sync_copy(y_hbm.at[row], acc.at[row])

        @pl.when(c != _FASTC)
        def _():
            zero16 = jnp.zeros((_LANES,), jnp.float32)

            def zr(r, u):
                for k in range(width // _LANES):
                    rows[0, r, pl.ds(k * _LANES, _LANES)] = zero16
                return u

            lax.fori_loop(0, 64, zr, 0)

            def zc(k, u):
                pltpu.sync_copy(
                    rows.at[0], acc.at[pl.ds(s * _RPT + k * 64, 64)]
                )
                return u

            lax.fori_loop(0, _RPT // 64, zc, 0)

        plsc.subcore_barrier()

        sems = (sem0, sem1)
        mask = jnp.full((_LANES,), 0xFFFF, jnp.int32)

        def unpack(i, b):
            for v in range(64 // _LANES):
                pk = packb[i, pl.ds(b * 64 + v * _LANES, _LANES)]
                sl = pl.ds(v * _LANES, _LANES)
                sidx[b, sl] = lax.bitwise_and(pk, mask)
                didx[b, sl] = lax.shift_right_logical(pk, 16)

        def gather_start(b):
            pltpu.async_copy(y_hbm.at[sidx.at[b]], rows.at[b], sems[b])

        def gather_wait(b):
            pltpu.make_async_copy(y_hbm.at[sidx.at[b]], rows.at[b], sems[b]).wait()

        def scatter(b):
            pltpu.sync_copy(rows.at[b], acc.at[didx.at[b]], add=True)

        for b in range(2):
            unpack(0, b)
            gather_start(b)

        def body(i, u):
            for b in range(2):
                gather_wait(b)
                scatter(b)
                unpack(i + 1, b)
                gather_start(b)
            return u

        lax.fori_loop(0, nch - 1, body, 0)

        for b in range(2):
            gather_wait(b)
            scatter(b)

        plsc.subcore_barrier()
        pltpu.sync_copy(acc.at[row], z_hbm.at[c, row])

    return _spmm


_spmm_h = _make_spmm(_H)


def _tc_pre_body(deg_ref, x_ref, w_ref, y_ref, dinv_ref):
    deg = deg_ref[0, :] + deg_ref[1, :] + 1.0
    dinv = lax.rsqrt(deg)
    dinv_ref[...] = dinv
    y_ref[...] = jnp.dot(
        x_ref[...] * dinv[:, None], w_ref[...], preferred_element_type=jnp.float32
    )


_tc_pre = pl.pallas_call(
    _tc_pre_body,
    out_shape=(
        jax.ShapeDtypeStruct((_NPAD, _H), jnp.float32),
        jax.ShapeDtypeStruct((_NPAD,), jnp.float32),
    ),
)


def _tc_mid1_body(z_ref, dinv_ref, b_ref, w_ref, h_ref, ynext_ref):
    dinv = dinv_ref[...]
    z = z_ref[0] + z_ref[1]
    t = z * dinv[:, None] + b_ref[...][None, :]
    h = jnp.where(t > 0, t, jnp.exp(jnp.minimum(t, 0.0)) - 1.0)
    h_ref[...] = h
    ynext_ref[...] = jnp.dot(
        h * dinv[:, None], w_ref[...], preferred_element_type=jnp.float32
    )


def _tc_mid2_body(z_ref, res_ref, dinv_ref, b_ref, h_ref, u_ref):
    dinv = dinv_ref[...]
    z = z_ref[0] + z_ref[1]
    t = z * dinv[:, None] + b_ref[...][None, :] + res_ref[...]
    h = jnp.where(t > 0, t, jnp.exp(jnp.minimum(t, 0.0)) - 1.0)
    h_ref[...] = h
    u_ref[...] = h * dinv[:, None]


_tc_mid1 = pl.pallas_call(
    _tc_mid1_body,
    out_shape=(
        jax.ShapeDtypeStruct((_NPAD, _H), jnp.float32),
        jax.ShapeDtypeStruct((_NPAD, _H), jnp.float32),
    ),
)

_tc_mid2 = pl.pallas_call(
    _tc_mid2_body,
    out_shape=(
        jax.ShapeDtypeStruct((_NPAD, _H), jnp.float32),
        jax.ShapeDtypeStruct((_NPAD, _H), jnp.float32),
    ),
)


def _tc_post_body(z_ref, dinv_ref, b_ref, w_ref, out_ref):
    z = z_ref[0] + z_ref[1]
    out_ref[...] = (
        jnp.dot(z * dinv_ref[...][:, None], w_ref[...],
                preferred_element_type=jnp.float32)
        + b_ref[...][None, :]
    )


_tc_post = pl.pallas_call(
    _tc_post_body,
    out_shape=jax.ShapeDtypeStruct((_NPAD, _CPAD), jnp.float32),
)


def kernel(x, edge_index, W1, b1, W2, b2, W3, b3):
    src = edge_index[0].astype(jnp.int32)
    dst = edge_index[1].astype(jnp.int32)
    packed = jnp.bitwise_or(src, jnp.left_shift(dst, 16))
    fillv = jnp.int32(_N | (_N << 16))
    fill = jnp.full((_EPAD - _E,), fillv, jnp.int32)
    packp = jnp.concatenate([packed, fill]).reshape(_NROW, 128)
    xp = jnp.zeros((_NPAD, _D), jnp.float32).at[:_N].set(x)
    w3p = jnp.zeros((_H, _CPAD), jnp.float32).at[:, :_C].set(W3)
    b3p = jnp.zeros((_CPAD,), jnp.float32).at[:_C].set(b3)

    deg = _deg_kernel(packp)
    y1, dinv = _tc_pre(deg, xp, W1)
    z1 = _spmm_h(y1, packp)
    h1, y2 = _tc_mid1(z1, dinv, b1, W2)
    z2 = _spmm_h(y2, packp)
    h2, u3 = _tc_mid2(z2, h1, dinv, b2)
    z3 = _spmm_h(u3, packp)
    out = _tc_post(z3, dinv, b3p, w3p)
    return out[:_N, :_C]
